# baseline (device time: 24981 ns/iter reference)
import jax
import jax.numpy as jnp
from jax import lax
from jax.experimental import pallas as pl
from jax.experimental.pallas import tpu as pltpu

M = 2048
N = 1024
HALF = 512
C = 8
CH = HALF // C
LAG = 2
EPS = 1e-6


def kernel(partial, gamma):
    g = gamma.reshape(1, N)

    def body(p_hbm, g_ref, out_ref, ps_f, pl_f, send_y, recv_y, fwd_x, recv_x,
             sem_cs, sem_cl, sem_sy, sem_ry, sem_sx, sem_rx):
        my_x = lax.axis_index("x")
        my_y = lax.axis_index("y")
        y_nbr = (my_x, 1 - my_y)
        x_nbr = (1 - my_x, my_y)
        send_base = (1 - my_y) * (M // 2) + my_x * HALF
        loc_base = my_y * (M // 2) + my_x * HALF

        cp_s = pltpu.make_async_copy(
            p_hbm.at[0, pl.ds(send_base, HALF), :], ps_f, sem_cs
        )
        cp_s.start()
        cp_l = pltpu.make_async_copy(
            p_hbm.at[0, pl.ds(loc_base, HALF), :], pl_f, sem_cl
        )
        cp_l.start()

        barrier = pltpu.get_barrier_semaphore()
        for nbr in (y_nbr, x_nbr):
            pl.semaphore_signal(
                barrier, inc=1, device_id=nbr,
                device_id_type=pl.DeviceIdType.MESH,
            )
        pl.semaphore_wait(barrier, 2)

        cp_s.wait()
        send_y[...] = ps_f[...].astype(jnp.bfloat16)
        y_rdmas = []
        for c in range(C):
            sl = pl.ds(c * CH, CH)
            r = pltpu.make_async_remote_copy(
                src_ref=send_y.at[sl, :],
                dst_ref=recv_y.at[sl, :],
                send_sem=sem_sy.at[c],
                recv_sem=sem_ry.at[c],
                device_id=y_nbr,
                device_id_type=pl.DeviceIdType.MESH,
            )
            r.start()
            y_rdmas.append(r)
        cp_l.wait()

        def norm_x_chunk(c):
            sl = pl.ds(c * CH, CH)
            x_rdmas[c].wait_recv()
            f = recv_x[sl, :].astype(jnp.float32)
            scale = lax.rsqrt(jnp.mean(f * f, axis=-1, keepdims=True) + EPS)
            out_ref[pl.ds((1 - my_x) * HALF + c * CH, CH), :] = (
                f * (scale * g_ref[...])
            ).astype(jnp.bfloat16)

        x_rdmas = []
        for c in range(C):
            sl = pl.ds(c * CH, CH)
            y_rdmas[c].wait_recv()
            f = recv_y[sl, :].astype(jnp.float32) + pl_f[sl, :]
            fwd_x[sl, :] = f.astype(jnp.bfloat16)
            r = pltpu.make_async_remote_copy(
                src_ref=fwd_x.at[sl, :],
                dst_ref=recv_x.at[sl, :],
                send_sem=sem_sx.at[c],
                recv_sem=sem_rx.at[c],
                device_id=x_nbr,
                device_id_type=pl.DeviceIdType.MESH,
            )
            r.start()
            x_rdmas.append(r)
            scale = lax.rsqrt(jnp.mean(f * f, axis=-1, keepdims=True) + EPS)
            out_ref[pl.ds(my_x * HALF + c * CH, CH), :] = (
                f * (scale * g_ref[...])
            ).astype(jnp.bfloat16)
            if c >= LAG:
                norm_x_chunk(c - LAG)

        for c in range(C - LAG, C):
            norm_x_chunk(c)

        for c in range(C):
            y_rdmas[c].wait_send()
            x_rdmas[c].wait_send()

    return pl.pallas_call(
        body,
        out_shape=jax.ShapeDtypeStruct((M // 2, N), jnp.bfloat16),
        in_specs=[
            pl.BlockSpec(memory_space=pltpu.MemorySpace.HBM),
            pl.BlockSpec(memory_space=pltpu.VMEM),
        ],
        out_specs=pl.BlockSpec(memory_space=pltpu.VMEM),
        scratch_shapes=[
            pltpu.VMEM((HALF, N), jnp.float32),
            pltpu.VMEM((HALF, N), jnp.float32),
            pltpu.VMEM((HALF, N), jnp.bfloat16),
            pltpu.VMEM((HALF, N), jnp.bfloat16),
            pltpu.VMEM((HALF, N), jnp.bfloat16),
            pltpu.VMEM((HALF, N), jnp.bfloat16),
            pltpu.SemaphoreType.DMA,
            pltpu.SemaphoreType.DMA,
            pltpu.SemaphoreType.DMA((C,)),
            pltpu.SemaphoreType.DMA((C,)),
            pltpu.SemaphoreType.DMA((C,)),
            pltpu.SemaphoreType.DMA((C,)),
        ],
        compiler_params=pltpu.CompilerParams(collective_id=0),
    )(partial, g)


# device time: 23990 ns/iter; 1.0413x vs baseline; 1.0413x over previous
import jax
import jax.numpy as jnp
from jax import lax
from jax.experimental import pallas as pl
from jax.experimental.pallas import tpu as pltpu

M = 2048
N = 1024
HALF = 512
C = 8
CH = HALF // C
EPS = 1e-6


def kernel(partial, gamma):
    my_x = lax.axis_index("x")
    my_y = lax.axis_index("y")
    send_base = (1 - my_y) * (M // 2) + my_x * HALF
    loc_base = my_y * (M // 2) + my_x * HALF
    p_send = lax.dynamic_slice(
        partial, (0, send_base, 0), (1, HALF, N)
    ).astype(jnp.bfloat16)
    p_loc = lax.dynamic_slice(
        partial, (0, loc_base, 0), (1, HALF, N)
    ).astype(jnp.bfloat16)
    g = gamma.reshape(1, N)

    def body(ps_ref, pl_ref, g_ref, out_ref, recv_y, fwd_x, recv_x,
             sem_sy, sem_ry, sem_sx, sem_rx):
        my_x = lax.axis_index("x")
        my_y = lax.axis_index("y")
        y_nbr = (my_x, 1 - my_y)
        x_nbr = (1 - my_x, my_y)

        barrier = pltpu.get_barrier_semaphore()
        for nbr in (y_nbr, x_nbr):
            pl.semaphore_signal(
                barrier, inc=1, device_id=nbr,
                device_id_type=pl.DeviceIdType.MESH,
            )
        pl.semaphore_wait(barrier, 2)

        y_rdmas = []
        for c in range(C):
            sl = pl.ds(c * CH, CH)
            r = pltpu.make_async_remote_copy(
                src_ref=ps_ref.at[0, sl, :],
                dst_ref=recv_y.at[sl, :],
                send_sem=sem_sy.at[c],
                recv_sem=sem_ry.at[c],
                device_id=y_nbr,
                device_id_type=pl.DeviceIdType.MESH,
            )
            r.start()
            y_rdmas.append(r)

        def norm_x_chunk(c):
            sl = pl.ds(c * CH, CH)
            x_rdmas[c].wait_recv()
            f = recv_x[sl, :].astype(jnp.float32)
            scale = lax.rsqrt(jnp.mean(f * f, axis=-1, keepdims=True) + EPS)
            out_ref[pl.ds((1 - my_x) * HALF + c * CH, CH), :] = (
                f * (scale * g_ref[...])
            ).astype(jnp.bfloat16)

        LAG = 2
        x_rdmas = []
        for c in range(C):
            sl = pl.ds(c * CH, CH)
            y_rdmas[c].wait_recv()
            s = recv_y[sl, :] + pl_ref[0, sl, :]
            fwd_x[sl, :] = s
            r = pltpu.make_async_remote_copy(
                src_ref=fwd_x.at[sl, :],
                dst_ref=recv_x.at[sl, :],
                send_sem=sem_sx.at[c],
                recv_sem=sem_rx.at[c],
                device_id=x_nbr,
                device_id_type=pl.DeviceIdType.MESH,
            )
            r.start()
            x_rdmas.append(r)
            f = s.astype(jnp.float32)
            scale = lax.rsqrt(jnp.mean(f * f, axis=-1, keepdims=True) + EPS)
            out_ref[pl.ds(my_x * HALF + c * CH, CH), :] = (
                f * (scale * g_ref[...])
            ).astype(jnp.bfloat16)
            if c >= LAG:
                norm_x_chunk(c - LAG)

        for c in range(C - LAG, C):
            norm_x_chunk(c)

        for c in range(C):
            y_rdmas[c].wait_send()
            x_rdmas[c].wait_send()

    return pl.pallas_call(
        body,
        out_shape=jax.ShapeDtypeStruct((M // 2, N), jnp.bfloat16),
        in_specs=[
            pl.BlockSpec(memory_space=pltpu.VMEM),
            pl.BlockSpec(memory_space=pltpu.VMEM),
            pl.BlockSpec(memory_space=pltpu.VMEM),
        ],
        out_specs=pl.BlockSpec(memory_space=pltpu.VMEM),
        scratch_shapes=[
            pltpu.VMEM((HALF, N), jnp.bfloat16),
            pltpu.VMEM((HALF, N), jnp.bfloat16),
            pltpu.VMEM((HALF, N), jnp.bfloat16),
            pltpu.SemaphoreType.DMA((C,)),
            pltpu.SemaphoreType.DMA((C,)),
            pltpu.SemaphoreType.DMA((C,)),
            pltpu.SemaphoreType.DMA((C,)),
        ],
        compiler_params=pltpu.CompilerParams(collective_id=0),
    )(p_send, p_loc, g)
